# baseline (speedup 1.0000x reference)
import functools

import jax
import jax.numpy as jnp
from jax import lax
from jax.experimental import pallas as pl
from jax.experimental.pallas import tpu as pltpu
from jax.experimental.pallas import tpu_sc as plsc

NC = 2
NS = 16
NW = NC * NS
CHUNK = 128
DEGW = 16


def _row_block(n):
    for blk in (1024, 1000, 512, 500, 256, 250, 128, 8):
        if n % blk == 0:
            return blk
    return n


@functools.lru_cache(maxsize=None)
def _build(N, E, C):
    mesh = plsc.VectorSubcoreMesh(core_axis_name="c", subcore_axis_name="s")

    nck = -(-E // (NW * CHUNK))
    nck += (-nck) % 2
    nblk = nck * NW
    e_pad = nblk * CHUNK
    n_pad = -(-N // (NS * 16)) * NS * 16
    if e_pad > E and n_pad == N:
        n_pad += NS * 16
    rpt = n_pad // NS

    @functools.partial(
        pl.kernel,
        out_type=jax.ShapeDtypeStruct((NC, n_pad, DEGW), jnp.float32),
        mesh=mesh,
        scratch_types=[
            pltpu.VMEM_SHARED((n_pad, DEGW), jnp.float32),
            pltpu.VMEM((2, CHUNK), jnp.int32),
            pltpu.VMEM((CHUNK, DEGW), jnp.float32),
            pltpu.VMEM((16, DEGW), jnp.float32),
        ],
    )
    def deg_kernel(ei_hbm, out_hbm, acc, idx_v, ones_v, zbuf):
        cid = lax.axis_index("c")
        sid = lax.axis_index("s")
        wid = cid * NS + sid
        base_row = sid * rpt

        @pl.loop(0, 16)
        def _(r):
            zbuf[r, :] = jnp.zeros((DEGW,), jnp.float32)

        @pl.loop(0, CHUNK)
        def _(r):
            ones_v[r, :] = jnp.ones((DEGW,), jnp.float32)

        @pl.loop(0, rpt, step=16)
        def _(r):
            pltpu.sync_copy(zbuf, acc.at[pl.ds(base_row + r, 16)])

        plsc.subcore_barrier()

        @pl.loop(0, nck)
        def _(k):
            pltpu.sync_copy(ei_hbm.at[wid * nck + k], idx_v)
            pltpu.sync_copy(ones_v, acc.at[idx_v.at[1]], add=True)

        plsc.subcore_barrier()
        pltpu.sync_copy(
            acc.at[pl.ds(base_row, rpt)],
            out_hbm.at[cid].at[pl.ds(base_row, rpt)],
        )

    @functools.partial(
        pl.kernel,
        out_type=jax.ShapeDtypeStruct((NC, n_pad, C), jnp.float32),
        mesh=mesh,
        scratch_types=[
            pltpu.VMEM_SHARED((n_pad, C), jnp.float32),
            pltpu.VMEM((2, CHUNK), jnp.int32),
            pltpu.VMEM((2, CHUNK), jnp.int32),
            pltpu.VMEM((CHUNK, C), jnp.float32),
            pltpu.VMEM((CHUNK, C), jnp.float32),
            pltpu.VMEM((16, C), jnp.float32),
            pltpu.SemaphoreType.DMA,
            pltpu.SemaphoreType.DMA,
        ],
    )
    def scatter_kernel(g_hbm, ei_hbm, out_hbm, acc, ib0, ib1,
                       rows0, rows1, zbuf, sg0, sg1):
        cid = lax.axis_index("c")
        sid = lax.axis_index("s")
        wid = cid * NS + sid
        base_row = sid * rpt
        ib = (ib0, ib1)
        rows = (rows0, rows1)
        sg = (sg0, sg1)
        cbase = wid * nck

        @pl.loop(0, 16)
        def _(r):
            @pl.loop(0, C, step=16)
            def _(j):
                zbuf[r, pl.ds(j, 16)] = jnp.zeros((16,), jnp.float32)

        @pl.loop(0, rpt, step=16)
        def _(r):
            pltpu.sync_copy(zbuf, acc.at[pl.ds(base_row + r, 16)])

        plsc.subcore_barrier()

        @pl.loop(0, nck)
        def _(kk):
            pltpu.sync_copy(ei_hbm.at[cbase + kk], ib[0])
            pltpu.sync_copy(g_hbm.at[ib[0].at[0]], rows[0])
            pltpu.sync_copy(rows[0], acc.at[ib[0].at[1]], add=True)

        plsc.subcore_barrier()
        pltpu.sync_copy(
            acc.at[pl.ds(base_row, rpt)],
            out_hbm.at[cid].at[pl.ds(base_row, rpt)],
        )

    blk = _row_block(N)
    grid = (N // blk,)

    def prep_body(x_ref, w_ref, degp_ref, g_ref):
        h = jnp.dot(x_ref[...], w_ref[...],
                    preferred_element_type=jnp.float32,
                    precision=lax.Precision.HIGHEST)
        deg = degp_ref[0, :, 0] + degp_ref[1, :, 0] + 1.0
        g_ref[...] = h * lax.rsqrt(deg)[:, None]

    prep = pl.pallas_call(
        prep_body,
        grid=grid,
        in_specs=[
            pl.BlockSpec((blk, C), lambda i: (i, 0)),
            pl.BlockSpec((C, C), lambda i: (0, 0)),
            pl.BlockSpec((NC, blk, DEGW), lambda i: (0, i, 0)),
        ],
        out_specs=pl.BlockSpec((blk, C), lambda i: (i, 0)),
        out_shape=jax.ShapeDtypeStruct((N, C), jnp.float32),
    )

    def final_body(degp_ref, p_ref, g_ref, b_ref, o_ref):
        deg = degp_ref[0, :, 0] + degp_ref[1, :, 0] + 1.0
        t = p_ref[0] + p_ref[1] + g_ref[...]
        o_ref[...] = t * lax.rsqrt(deg)[:, None] + b_ref[...]

    final = pl.pallas_call(
        final_body,
        grid=grid,
        in_specs=[
            pl.BlockSpec((NC, blk, DEGW), lambda i: (0, i, 0)),
            pl.BlockSpec((NC, blk, C), lambda i: (0, i, 0)),
            pl.BlockSpec((blk, C), lambda i: (i, 0)),
            pl.BlockSpec((1, C), lambda i: (0, 0)),
        ],
        out_specs=pl.BlockSpec((blk, C), lambda i: (i, 0)),
        out_shape=jax.ShapeDtypeStruct((N, C), jnp.float32),
    )

    return nblk, e_pad, deg_kernel, scatter_kernel, prep, final


def kernel(x, edge_index, W, b):
    N, C = x.shape
    E = edge_index.shape[1]
    nblk, e_pad, deg_kernel, scatter_kernel, prep, final = _build(N, E, C)

    src = edge_index[0].astype(jnp.int32)
    dst = edge_index[1].astype(jnp.int32)
    if e_pad > E:
        pad = jnp.arange(e_pad - E, dtype=jnp.int32) % 16
        src = jnp.concatenate([src, pad])
        dst = jnp.concatenate([dst, N + pad])
    ei = jnp.stack([src, dst]).reshape(2, nblk, CHUNK).transpose(1, 0, 2)

    degp = deg_kernel(ei)
    g = prep(x, W, degp)
    parts = scatter_kernel(g, ei)
    return final(degp, parts, g, b.reshape(1, C))


# speedup vs baseline: 1.1321x; 1.1321x over previous
import functools

import jax
import jax.numpy as jnp
from jax import lax
from jax.experimental import pallas as pl
from jax.experimental.pallas import tpu as pltpu
from jax.experimental.pallas import tpu_sc as plsc

NC = 2
NS = 16
NW = NC * NS
CHUNK = 128
DEGW = 16


def _row_block(n):
    for blk in (1024, 1000, 512, 500, 256, 250, 128, 8):
        if n % blk == 0:
            return blk
    return n


@functools.lru_cache(maxsize=None)
def _build(N, E, C):
    mesh = plsc.VectorSubcoreMesh(core_axis_name="c", subcore_axis_name="s")

    nck = -(-E // (NW * CHUNK))
    nck += nck % 2
    nblk = nck * NW
    e_pad = nblk * CHUNK
    n_pad = -(-N // (NS * 16)) * NS * 16
    if e_pad > E and n_pad == N:
        n_pad += NS * 16
    rpt = n_pad // NS

    @functools.partial(
        pl.kernel,
        out_type=jax.ShapeDtypeStruct((NC, n_pad, DEGW), jnp.float32),
        mesh=mesh,
        scratch_types=[
            pltpu.VMEM_SHARED((n_pad, DEGW), jnp.float32),
            pltpu.VMEM((2, 2, CHUNK), jnp.int32),
            pltpu.VMEM((CHUNK, DEGW), jnp.float32),
            pltpu.VMEM((CHUNK, DEGW), jnp.float32),
        ],
    )
    def deg_kernel(ei_hbm, out_hbm, acc, idx_v, ones_v, zbuf):
        cid = lax.axis_index("c")
        sid = lax.axis_index("s")
        wid = cid * NS + sid
        base_row = sid * rpt

        @pl.loop(0, CHUNK)
        def _(r):
            zbuf[r, :] = jnp.zeros((DEGW,), jnp.float32)
            ones_v[r, :] = jnp.ones((DEGW,), jnp.float32)

        @pl.loop(0, rpt, step=CHUNK)
        def _(r):
            pltpu.sync_copy(zbuf, acc.at[pl.ds(base_row + r, CHUNK)])

        plsc.subcore_barrier()

        @pl.loop(0, nck, step=2)
        def _(k):
            pltpu.sync_copy(ei_hbm.at[pl.ds(wid * nck + k, 2)], idx_v)
            pltpu.sync_copy(ones_v, acc.at[idx_v.at[0].at[1]], add=True)
            pltpu.sync_copy(ones_v, acc.at[idx_v.at[1].at[1]], add=True)

        plsc.subcore_barrier()
        pltpu.sync_copy(
            acc.at[pl.ds(base_row, rpt)],
            out_hbm.at[cid].at[pl.ds(base_row, rpt)],
        )

    @functools.partial(
        pl.kernel,
        out_type=jax.ShapeDtypeStruct((NC, n_pad, C), jnp.float32),
        mesh=mesh,
        scratch_types=[
            pltpu.VMEM_SHARED((n_pad, C), jnp.float32),
            pltpu.VMEM((2, 2, CHUNK), jnp.int32),
            pltpu.VMEM((CHUNK, C), jnp.float32),
        ],
    )
    def scatter_kernel(g_hbm, ei_hbm, out_hbm, acc, idx_v, rows_v):
        cid = lax.axis_index("c")
        sid = lax.axis_index("s")
        wid = cid * NS + sid
        base_row = sid * rpt

        @pl.loop(0, CHUNK)
        def _(r):
            @pl.loop(0, C, step=16)
            def _(j):
                rows_v[r, pl.ds(j, 16)] = jnp.zeros((16,), jnp.float32)

        @pl.loop(0, rpt, step=CHUNK)
        def _(r):
            pltpu.sync_copy(rows_v, acc.at[pl.ds(base_row + r, CHUNK)])

        plsc.subcore_barrier()

        @pl.loop(0, nck, step=2)
        def _(k):
            pltpu.sync_copy(ei_hbm.at[pl.ds(wid * nck + k, 2)], idx_v)
            pltpu.sync_copy(g_hbm.at[idx_v.at[0].at[0]], rows_v)
            pltpu.sync_copy(rows_v, acc.at[idx_v.at[0].at[1]], add=True)
            pltpu.sync_copy(g_hbm.at[idx_v.at[1].at[0]], rows_v)
            pltpu.sync_copy(rows_v, acc.at[idx_v.at[1].at[1]], add=True)

        plsc.subcore_barrier()
        pltpu.sync_copy(
            acc.at[pl.ds(base_row, rpt)],
            out_hbm.at[cid].at[pl.ds(base_row, rpt)],
        )

    blk = _row_block(N)
    grid = (N // blk,)

    def prep_body(x_ref, w_ref, degp_ref, g_ref):
        h = jnp.dot(x_ref[...], w_ref[...],
                    preferred_element_type=jnp.float32,
                    precision=lax.Precision.HIGHEST)
        deg = degp_ref[0, :, 0] + degp_ref[1, :, 0] + 1.0
        g_ref[...] = h * lax.rsqrt(deg)[:, None]

    prep = pl.pallas_call(
        prep_body,
        grid=grid,
        in_specs=[
            pl.BlockSpec((blk, C), lambda i: (i, 0)),
            pl.BlockSpec((C, C), lambda i: (0, 0)),
            pl.BlockSpec((NC, blk, DEGW), lambda i: (0, i, 0)),
        ],
        out_specs=pl.BlockSpec((blk, C), lambda i: (i, 0)),
        out_shape=jax.ShapeDtypeStruct((N, C), jnp.float32),
    )

    def final_body(degp_ref, p_ref, g_ref, b_ref, o_ref):
        deg = degp_ref[0, :, 0] + degp_ref[1, :, 0] + 1.0
        t = p_ref[0] + p_ref[1] + g_ref[...]
        o_ref[...] = t * lax.rsqrt(deg)[:, None] + b_ref[...]

    final = pl.pallas_call(
        final_body,
        grid=grid,
        in_specs=[
            pl.BlockSpec((NC, blk, DEGW), lambda i: (0, i, 0)),
            pl.BlockSpec((NC, blk, C), lambda i: (0, i, 0)),
            pl.BlockSpec((blk, C), lambda i: (i, 0)),
            pl.BlockSpec((1, C), lambda i: (0, 0)),
        ],
        out_specs=pl.BlockSpec((blk, C), lambda i: (i, 0)),
        out_shape=jax.ShapeDtypeStruct((N, C), jnp.float32),
    )

    return nblk, e_pad, deg_kernel, scatter_kernel, prep, final


def kernel(x, edge_index, W, b):
    N, C = x.shape
    E = edge_index.shape[1]
    nblk, e_pad, deg_kernel, scatter_kernel, prep, final = _build(N, E, C)

    src = edge_index[0].astype(jnp.int32)
    dst = edge_index[1].astype(jnp.int32)
    if e_pad > E:
        pad = jnp.arange(e_pad - E, dtype=jnp.int32) % 16
        src = jnp.concatenate([src, pad])
        dst = jnp.concatenate([dst, N + pad])
    ei = jnp.stack([src, dst]).reshape(2, nblk, CHUNK).transpose(1, 0, 2)

    degp = deg_kernel(ei)
    g = prep(x, W, degp)
    parts = scatter_kernel(g, ei)
    return final(degp, parts, g, b.reshape(1, C))


# speedup vs baseline: 1.3276x; 1.1727x over previous
import functools

import jax
import jax.numpy as jnp
from jax import lax
from jax.experimental import pallas as pl
from jax.experimental.pallas import tpu as pltpu
from jax.experimental.pallas import tpu_sc as plsc

NC = 2
NS = 16
NW = NC * NS
CHUNK = 128
DEGW = 16


def _row_block(n):
    for blk in (1024, 1000, 512, 500, 256, 250, 128, 8):
        if n % blk == 0:
            return blk
    return n


@functools.lru_cache(maxsize=None)
def _build(N, E, C):
    mesh = plsc.VectorSubcoreMesh(core_axis_name="c", subcore_axis_name="s")

    nck = -(-E // (NW * CHUNK))
    nck += nck % 2
    nblk = nck * NW
    e_pad = nblk * CHUNK
    n_pad = -(-N // (NS * 16)) * NS * 16
    if e_pad > E and n_pad == N:
        n_pad += NS * 16
    rpt = n_pad // NS

    @functools.partial(
        pl.kernel,
        out_type=jax.ShapeDtypeStruct((NC, n_pad, DEGW), jnp.float32),
        mesh=mesh,
        scratch_types=[
            pltpu.VMEM_SHARED((n_pad, DEGW), jnp.float32),
            pltpu.VMEM((4, 2, CHUNK), jnp.int32),
            pltpu.VMEM((CHUNK, DEGW), jnp.float32),
            pltpu.VMEM((CHUNK, DEGW), jnp.float32),
        ],
    )
    def deg_kernel(ei_hbm, out_hbm, acc, idx_v, ones_v, zbuf):
        cid = lax.axis_index("c")
        sid = lax.axis_index("s")
        wid = cid * NS + sid
        base_row = sid * rpt

        @pl.loop(0, CHUNK)
        def _(r):
            zbuf[r, :] = jnp.zeros((DEGW,), jnp.float32)
            ones_v[r, :] = jnp.ones((DEGW,), jnp.float32)

        @pl.loop(0, rpt, step=CHUNK)
        def _(r):
            pltpu.sync_copy(zbuf, acc.at[pl.ds(base_row + r, CHUNK)])

        plsc.subcore_barrier()

        @pl.loop(0, nck, step=4)
        def _(k):
            pltpu.sync_copy(ei_hbm.at[pl.ds(wid * nck + k, 4)], idx_v)
            for t in range(4):
                pltpu.sync_copy(ones_v, acc.at[idx_v.at[t].at[1]], add=True)

        plsc.subcore_barrier()
        pltpu.sync_copy(
            acc.at[pl.ds(base_row, rpt)],
            out_hbm.at[cid].at[pl.ds(base_row, rpt)],
        )

    @functools.partial(
        pl.kernel,
        out_type=jax.ShapeDtypeStruct((NC, n_pad, C), jnp.float32),
        mesh=mesh,
        scratch_types=[
            pltpu.VMEM_SHARED((n_pad, C), jnp.float32),
            pltpu.VMEM((4, 2, CHUNK), jnp.int32),
            pltpu.VMEM((CHUNK, C), jnp.float32),
            pltpu.VMEM((CHUNK, C), jnp.float32),
            pltpu.SemaphoreType.DMA,
            pltpu.SemaphoreType.DMA,
        ],
    )
    def scatter_kernel(g_hbm, ei_hbm, out_hbm, acc, idx_v, rows_v, rows_w,
                       sg0, sg1):
        cid = lax.axis_index("c")
        sid = lax.axis_index("s")
        wid = cid * NS + sid
        base_row = sid * rpt

        @pl.loop(0, CHUNK)
        def _(r):
            @pl.loop(0, C, step=16)
            def _(j):
                rows_v[r, pl.ds(j, 16)] = jnp.zeros((16,), jnp.float32)

        @pl.loop(0, rpt, step=CHUNK)
        def _(r):
            pltpu.sync_copy(rows_v, acc.at[pl.ds(base_row + r, CHUNK)])

        plsc.subcore_barrier()

        @pl.loop(0, nck, step=4)
        def _(k):
            pltpu.sync_copy(ei_hbm.at[pl.ds(wid * nck + k, 4)], idx_v)
            for t in (0, 2):
                d0 = pltpu.async_copy(
                    g_hbm.at[idx_v.at[t].at[0]], rows_v, sg0)
                d1 = pltpu.async_copy(
                    g_hbm.at[idx_v.at[t + 1].at[0]], rows_w, sg1)
                d0.wait()
                pltpu.sync_copy(rows_v, acc.at[idx_v.at[t].at[1]], add=True)
                d1.wait()
                pltpu.sync_copy(rows_w, acc.at[idx_v.at[t + 1].at[1]], add=True)

        plsc.subcore_barrier()
        pltpu.sync_copy(
            acc.at[pl.ds(base_row, rpt)],
            out_hbm.at[cid].at[pl.ds(base_row, rpt)],
        )

    blk = _row_block(N)
    grid = (N // blk,)

    def prep_body(x_ref, w_ref, degp_ref, g_ref):
        h = jnp.dot(x_ref[...], w_ref[...],
                    preferred_element_type=jnp.float32,
                    precision=lax.Precision.HIGHEST)
        deg = degp_ref[0, :, 0] + degp_ref[1, :, 0] + 1.0
        g_ref[...] = h * lax.rsqrt(deg)[:, None]

    prep = pl.pallas_call(
        prep_body,
        grid=grid,
        in_specs=[
            pl.BlockSpec((blk, C), lambda i: (i, 0)),
            pl.BlockSpec((C, C), lambda i: (0, 0)),
            pl.BlockSpec((NC, blk, DEGW), lambda i: (0, i, 0)),
        ],
        out_specs=pl.BlockSpec((blk, C), lambda i: (i, 0)),
        out_shape=jax.ShapeDtypeStruct((N, C), jnp.float32),
    )

    def final_body(degp_ref, p_ref, g_ref, b_ref, o_ref):
        deg = degp_ref[0, :, 0] + degp_ref[1, :, 0] + 1.0
        t = p_ref[0] + p_ref[1] + g_ref[...]
        o_ref[...] = t * lax.rsqrt(deg)[:, None] + b_ref[...]

    final = pl.pallas_call(
        final_body,
        grid=grid,
        in_specs=[
            pl.BlockSpec((NC, blk, DEGW), lambda i: (0, i, 0)),
            pl.BlockSpec((NC, blk, C), lambda i: (0, i, 0)),
            pl.BlockSpec((blk, C), lambda i: (i, 0)),
            pl.BlockSpec((1, C), lambda i: (0, 0)),
        ],
        out_specs=pl.BlockSpec((blk, C), lambda i: (i, 0)),
        out_shape=jax.ShapeDtypeStruct((N, C), jnp.float32),
    )

    return nblk, e_pad, deg_kernel, scatter_kernel, prep, final


def kernel(x, edge_index, W, b):
    N, C = x.shape
    E = edge_index.shape[1]
    nblk, e_pad, deg_kernel, scatter_kernel, prep, final = _build(N, E, C)

    src = edge_index[0].astype(jnp.int32)
    dst = edge_index[1].astype(jnp.int32)
    if e_pad > E:
        pad = jnp.arange(e_pad - E, dtype=jnp.int32) % 16
        src = jnp.concatenate([src, pad])
        dst = jnp.concatenate([dst, N + pad])
    ei = jnp.stack([src, dst]).reshape(2, nblk, CHUNK).transpose(1, 0, 2)

    degp = deg_kernel(ei)
    g = prep(x, W, degp)
    parts = scatter_kernel(g, ei)
    return final(degp, parts, g, b.reshape(1, C))


# speedup vs baseline: 1.4667x; 1.1048x over previous
import functools

import jax
import jax.numpy as jnp
from jax import lax
from jax.experimental import pallas as pl
from jax.experimental.pallas import tpu as pltpu
from jax.experimental.pallas import tpu_sc as plsc

NC = 2
NS = 16
NW = NC * NS
CHUNK = 128
DEGW = 16


def _row_block(n):
    for blk in (1024, 1000, 512, 500, 256, 250, 128, 8):
        if n % blk == 0:
            return blk
    return n


@functools.lru_cache(maxsize=None)
def _build(N, E, C):
    mesh = plsc.VectorSubcoreMesh(core_axis_name="c", subcore_axis_name="s")

    nck = -(-E // (NW * CHUNK))
    nck += (-nck) % 8
    nblk = nck * NW
    e_pad = nblk * CHUNK
    n_pad = -(-N // (NS * 16)) * NS * 16
    if e_pad > E and n_pad == N:
        n_pad += NS * 16
    rpt = n_pad // NS

    @functools.partial(
        pl.kernel,
        out_type=jax.ShapeDtypeStruct((NC, n_pad, DEGW), jnp.float32),
        mesh=mesh,
        scratch_types=[
            pltpu.VMEM_SHARED((n_pad, DEGW), jnp.float32),
            pltpu.VMEM((4, 2, CHUNK), jnp.int32),
            pltpu.VMEM((CHUNK, DEGW), jnp.float32),
            pltpu.VMEM((CHUNK, DEGW), jnp.float32),
        ],
    )
    def deg_kernel(ei_hbm, out_hbm, acc, idx_v, ones_v, zbuf):
        cid = lax.axis_index("c")
        sid = lax.axis_index("s")
        wid = cid * NS + sid
        base_row = sid * rpt

        @pl.loop(0, CHUNK)
        def _(r):
            zbuf[r, :] = jnp.zeros((DEGW,), jnp.float32)
            ones_v[r, :] = jnp.ones((DEGW,), jnp.float32)

        @pl.loop(0, rpt, step=CHUNK)
        def _(r):
            pltpu.sync_copy(zbuf, acc.at[pl.ds(base_row + r, CHUNK)])

        plsc.subcore_barrier()

        @pl.loop(0, nck, step=4)
        def _(k):
            pltpu.sync_copy(ei_hbm.at[pl.ds(wid * nck + k, 4)], idx_v)
            for t in range(4):
                pltpu.sync_copy(ones_v, acc.at[idx_v.at[t].at[1]], add=True)

        plsc.subcore_barrier()
        pltpu.sync_copy(
            acc.at[pl.ds(base_row, rpt)],
            out_hbm.at[cid].at[pl.ds(base_row, rpt)],
        )

    @functools.partial(
        pl.kernel,
        out_type=jax.ShapeDtypeStruct((NC, n_pad, C), jnp.float32),
        mesh=mesh,
        scratch_types=[
            pltpu.VMEM_SHARED((n_pad, C), jnp.float32),
            pltpu.VMEM((8, 2, CHUNK), jnp.int32),
            pltpu.VMEM((CHUNK, C), jnp.float32),
            pltpu.VMEM((CHUNK, C), jnp.float32),
            pltpu.SemaphoreType.DMA,
            pltpu.SemaphoreType.DMA,
            pltpu.SemaphoreType.DMA,
            pltpu.SemaphoreType.DMA,
        ],
    )
    def scatter_kernel(g_hbm, ei_hbm, out_hbm, acc, idx_v, rows_v, rows_w,
                       sg0, sg1, sb0, sb1):
        cid = lax.axis_index("c")
        sid = lax.axis_index("s")
        wid = cid * NS + sid
        base_row = sid * rpt

        @pl.loop(0, CHUNK)
        def _(r):
            @pl.loop(0, C, step=16)
            def _(j):
                rows_v[r, pl.ds(j, 16)] = jnp.zeros((16,), jnp.float32)

        @pl.loop(0, rpt, step=CHUNK)
        def _(r):
            pltpu.sync_copy(rows_v, acc.at[pl.ds(base_row + r, CHUNK)])

        plsc.subcore_barrier()

        rows = (rows_v, rows_w)
        sg = (sg0, sg1)
        sb = (sb0, sb1)

        @pl.loop(0, nck, step=8)
        def _(k):
            pltpu.sync_copy(ei_hbm.at[pl.ds(wid * nck + k, 8)], idx_v)
            a = [None] * 8
            b = [None] * 8
            a[0] = pltpu.async_copy(g_hbm.at[idx_v.at[0].at[0]], rows[0], sg[0])
            a[1] = pltpu.async_copy(g_hbm.at[idx_v.at[1].at[0]], rows[1], sg[1])
            for t in range(8):
                p = t % 2
                if t >= 2:
                    b[t - 2].wait()
                    a[t] = pltpu.async_copy(
                        g_hbm.at[idx_v.at[t].at[0]], rows[p], sg[p])
                a[t].wait()
                b[t] = pltpu.async_copy(
                    rows[p], acc.at[idx_v.at[t].at[1]], sb[p], add=True)
            b[6].wait()
            b[7].wait()

        plsc.subcore_barrier()
        pltpu.sync_copy(
            acc.at[pl.ds(base_row, rpt)],
            out_hbm.at[cid].at[pl.ds(base_row, rpt)],
        )

    blk = _row_block(N)
    grid = (N // blk,)

    def prep_body(x_ref, w_ref, degp_ref, g_ref):
        h = jnp.dot(x_ref[...], w_ref[...],
                    preferred_element_type=jnp.float32,
                    precision=lax.Precision.HIGHEST)
        deg = degp_ref[0, :, 0] + degp_ref[1, :, 0] + 1.0
        g_ref[...] = h * lax.rsqrt(deg)[:, None]

    prep = pl.pallas_call(
        prep_body,
        grid=grid,
        in_specs=[
            pl.BlockSpec((blk, C), lambda i: (i, 0)),
            pl.BlockSpec((C, C), lambda i: (0, 0)),
            pl.BlockSpec((NC, blk, DEGW), lambda i: (0, i, 0)),
        ],
        out_specs=pl.BlockSpec((blk, C), lambda i: (i, 0)),
        out_shape=jax.ShapeDtypeStruct((N, C), jnp.float32),
    )

    def final_body(degp_ref, p_ref, g_ref, b_ref, o_ref):
        deg = degp_ref[0, :, 0] + degp_ref[1, :, 0] + 1.0
        t = p_ref[0] + p_ref[1] + g_ref[...]
        o_ref[...] = t * lax.rsqrt(deg)[:, None] + b_ref[...]

    final = pl.pallas_call(
        final_body,
        grid=grid,
        in_specs=[
            pl.BlockSpec((NC, blk, DEGW), lambda i: (0, i, 0)),
            pl.BlockSpec((NC, blk, C), lambda i: (0, i, 0)),
            pl.BlockSpec((blk, C), lambda i: (i, 0)),
            pl.BlockSpec((1, C), lambda i: (0, 0)),
        ],
        out_specs=pl.BlockSpec((blk, C), lambda i: (i, 0)),
        out_shape=jax.ShapeDtypeStruct((N, C), jnp.float32),
    )

    return nblk, e_pad, deg_kernel, scatter_kernel, prep, final


def kernel(x, edge_index, W, b):
    N, C = x.shape
    E = edge_index.shape[1]
    nblk, e_pad, deg_kernel, scatter_kernel, prep, final = _build(N, E, C)

    src = edge_index[0].astype(jnp.int32)
    dst = edge_index[1].astype(jnp.int32)
    if e_pad > E:
        pad = jnp.arange(e_pad - E, dtype=jnp.int32) % 16
        src = jnp.concatenate([src, pad])
        dst = jnp.concatenate([dst, N + pad])
    ei = jnp.stack([src, dst]).reshape(2, nblk, CHUNK).transpose(1, 0, 2)

    degp = deg_kernel(ei)
    g = prep(x, W, degp)
    parts = scatter_kernel(g, ei)
    return final(degp, parts, g, b.reshape(1, C))
